# baseline (speedup 1.0000x reference)
import functools

import jax
import jax.numpy as jnp
from jax import lax
from jax.experimental import pallas as pl
from jax.experimental.pallas import tpu as pltpu
from jax.experimental.pallas import tpu_sc as plsc

_P, _S, _NC, _D = 3, 512, 1024, 64
_TEMP = 0.07
_MATCH_W = 0.2
_CONF_W = 0.02
_B = _P * _S
_V = _P * _NC
_W = 128

_NCORES = 2
_NSUB = 16
_NW = _NCORES * _NSUB
_CPW = _S // _NW
_BPW = _P * _CPW


def _sc_gather_body(dsp_hbm, dtp_hbm, si_hbm, ti_hbm, q_out, t_out,
                    si_v, ti_v, q_rows, t_rows, sem_i, sem_g, sem_o):
    wid = lax.axis_index("s") * _NCORES + lax.axis_index("c")
    c0 = wid * _CPW

    loads = []
    for p in range(_P):
        loads.append(pltpu.async_copy(
            si_hbm.at[p, pl.ds(c0, _CPW)], si_v.at[pl.ds(p * _CPW, _CPW)], sem_i))
        loads.append(pltpu.async_copy(
            ti_hbm.at[p, pl.ds(c0, _CPW)], ti_v.at[pl.ds(p * _CPW, _CPW)], sem_i))
    for ld in loads:
        ld.wait()

    for p in range(1, _P):
        si_v[pl.ds(p * _CPW, _CPW)] = si_v[pl.ds(p * _CPW, _CPW)] + (p * _NC)
        ti_v[pl.ds(p * _CPW, _CPW)] = ti_v[pl.ds(p * _CPW, _CPW)] + (p * _NC)

    g_q = pltpu.async_copy(dsp_hbm.at[si_v], q_rows, sem_g)
    g_t = pltpu.async_copy(dtp_hbm.at[ti_v], t_rows, sem_g)

    stores = []
    g_q.wait()
    for p in range(_P):
        stores.append(pltpu.async_copy(
            q_rows.at[pl.ds(p * _CPW, _CPW)], q_out.at[p, pl.ds(c0, _CPW)], sem_o))
    g_t.wait()
    for p in range(_P):
        stores.append(pltpu.async_copy(
            t_rows.at[pl.ds(p * _CPW, _CPW)], t_out.at[p, pl.ds(c0, _CPW)], sem_o))
    for st in stores:
        st.wait()


@functools.cache
def _make_sc_gather():
    return functools.partial(
        pl.kernel,
        out_type=(
            jax.ShapeDtypeStruct((_P, _S, _W), jnp.float32),
            jax.ShapeDtypeStruct((_P, _S, _W), jnp.float32),
        ),
        mesh=plsc.VectorSubcoreMesh(core_axis_name="c", subcore_axis_name="s"),
        scratch_types=[
            pltpu.VMEM((_BPW,), jnp.int32),
            pltpu.VMEM((_BPW,), jnp.int32),
            pltpu.VMEM((_BPW, _W), jnp.float32),
            pltpu.VMEM((_BPW, _W), jnp.float32),
            pltpu.SemaphoreType.DMA,
            pltpu.SemaphoreType.DMA,
            pltpu.SemaphoreType.DMA,
        ],
    )(_sc_gather_body)


def _norm_rows(x):
    return x / (jnp.sqrt(jnp.sum(x * x, axis=-1, keepdims=True)) + 1e-6)


def _bce(logit, target):
    return (jnp.maximum(logit, 0.0) - logit * target
            + jnp.log1p(jnp.exp(-jnp.abs(logit))))


def _tc_body(ds_ref, dt_ref, q_ref, t_ref, valid_ref, out_ref):
    inv_t = 1.0 / _TEMP
    dn = (((1,), (1,)), ((), ()))
    tot_ce = 0.0
    tot_vm = 0.0
    tot_conf = 0.0
    for p in range(_P):
        dsn = _norm_rows(ds_ref[p])
        dtn = _norm_rows(dt_ref[p])
        qrow = q_ref[p]
        trow = t_ref[p]
        qn = _norm_rows(qrow[:, :_D])
        tn = _norm_rows(trow[:, :_D])
        cs = jnp.sum(qrow[:, _D:], axis=-1)
        ct = jnp.sum(trow[:, _D:], axis=-1)
        vmask = (valid_ref[p] > 0.5).astype(jnp.float32)

        qb = (qn * inv_t).astype(jnp.bfloat16)
        tb = (tn * inv_t).astype(jnp.bfloat16)
        l_st = lax.dot_general(qb, dtn.astype(jnp.bfloat16), dn,
                               preferred_element_type=jnp.float32)
        lse_st = jnp.log(jnp.sum(jnp.exp(l_st), axis=-1))
        l_ts = lax.dot_general(tb, dsn.astype(jnp.bfloat16), dn,
                               preferred_element_type=jnp.float32)
        lse_ts = jnp.log(jnp.sum(jnp.exp(l_ts), axis=-1))
        gold = jnp.sum(qn * tn, axis=-1) * inv_t

        ce_pair = (lse_st + lse_ts - 2.0 * gold) * 0.5
        tot_ce += jnp.sum(ce_pair * vmask)
        tot_vm += jnp.sum(vmask)
        tot_conf += jnp.sum((_bce(cs, vmask) + _bce(ct, vmask)) * 0.5)

    coarse = tot_ce / jnp.maximum(tot_vm, 1.0)
    out_ref[0, 0] = _MATCH_W * coarse + _CONF_W * (tot_conf / float(_B))


_tc_loss = pl.pallas_call(
    _tc_body,
    out_specs=pl.BlockSpec(memory_space=pltpu.SMEM),
    out_shape=jax.ShapeDtypeStruct((1, 1), jnp.float32),
)


def kernel(desc_src, desc_tgt, conf_src, conf_tgt, src_idx, tgt_idx, valid):
    packed_s = (jnp.pad(desc_src, ((0, 0), (0, 0), (0, _W - _D)))
                + jnp.pad(conf_src[..., None], ((0, 0), (0, 0), (_D, _W - _D - 1))))
    packed_t = (jnp.pad(desc_tgt, ((0, 0), (0, 0), (0, _W - _D)))
                + jnp.pad(conf_tgt[..., None], ((0, 0), (0, 0), (_D, _W - _D - 1))))
    q_rows, t_rows = _make_sc_gather()(
        packed_s.reshape(_V, _W), packed_t.reshape(_V, _W),
        src_idx.astype(jnp.int32), tgt_idx.astype(jnp.int32))
    out = _tc_loss(desc_src, desc_tgt, q_rows, t_rows, valid)
    return out[0, 0]


# speedup vs baseline: 1.0172x; 1.0172x over previous
import functools

import jax
import jax.numpy as jnp
from jax import lax
from jax.experimental import pallas as pl
from jax.experimental.pallas import tpu as pltpu
from jax.experimental.pallas import tpu_sc as plsc

_P, _S, _NC, _D = 3, 512, 1024, 64
_TEMP = 0.07
_MATCH_W = 0.2
_CONF_W = 0.02
_B = _P * _S
_V = _P * _NC
_W = 128

_NCORES = 2
_NSUB = 16
_NW = _NCORES * _NSUB
_CPW = _S // _NW
_BPW = _P * _CPW


def _sc_gather_body(dsp_hbm, dtp_hbm, si_hbm, ti_hbm, q_out, t_out,
                    si_v, ti_v, q_rows, t_rows, sem_i, sem_g, sem_o):
    wid = lax.axis_index("s") * _NCORES + lax.axis_index("c")
    c0 = wid * _CPW

    loads = []
    for p in range(_P):
        loads.append(pltpu.async_copy(
            si_hbm.at[p, pl.ds(c0, _CPW)], si_v.at[pl.ds(p * _CPW, _CPW)], sem_i))
        loads.append(pltpu.async_copy(
            ti_hbm.at[p, pl.ds(c0, _CPW)], ti_v.at[pl.ds(p * _CPW, _CPW)], sem_i))
    for ld in loads:
        ld.wait()

    for p in range(1, _P):
        si_v[pl.ds(p * _CPW, _CPW)] = si_v[pl.ds(p * _CPW, _CPW)] + (p * _NC)
        ti_v[pl.ds(p * _CPW, _CPW)] = ti_v[pl.ds(p * _CPW, _CPW)] + (p * _NC)

    g_q = pltpu.async_copy(dsp_hbm.at[si_v], q_rows, sem_g)
    g_t = pltpu.async_copy(dtp_hbm.at[ti_v], t_rows, sem_g)

    stores = []
    g_q.wait()
    for p in range(_P):
        stores.append(pltpu.async_copy(
            q_rows.at[pl.ds(p * _CPW, _CPW)], q_out.at[p, pl.ds(c0, _CPW)], sem_o))
    g_t.wait()
    for p in range(_P):
        stores.append(pltpu.async_copy(
            t_rows.at[pl.ds(p * _CPW, _CPW)], t_out.at[p, pl.ds(c0, _CPW)], sem_o))
    for st in stores:
        st.wait()


@functools.cache
def _make_sc_gather():
    return functools.partial(
        pl.kernel,
        out_type=(
            jax.ShapeDtypeStruct((_P, _S, _W), jnp.float32),
            jax.ShapeDtypeStruct((_P, _S, _W), jnp.float32),
        ),
        mesh=plsc.VectorSubcoreMesh(core_axis_name="c", subcore_axis_name="s"),
        scratch_types=[
            pltpu.VMEM((_BPW,), jnp.int32),
            pltpu.VMEM((_BPW,), jnp.int32),
            pltpu.VMEM((_BPW, _W), jnp.float32),
            pltpu.VMEM((_BPW, _W), jnp.float32),
            pltpu.SemaphoreType.DMA,
            pltpu.SemaphoreType.DMA,
            pltpu.SemaphoreType.DMA,
        ],
    )(_sc_gather_body)


def _norm_rows(x):
    return x / (jnp.sqrt(jnp.sum(x * x, axis=-1, keepdims=True)) + 1e-6)


def _bce(logit, target):
    return (jnp.maximum(logit, 0.0) - logit * target
            + jnp.log1p(jnp.exp(-jnp.abs(logit))))


def _tc_body(ds_ref, dt_ref, q_ref, t_ref, valid_ref, out_ref, acc):
    p = pl.program_id(0)

    @pl.when(p == 0)
    def _():
        acc[0] = 0.0
        acc[1] = 0.0
        acc[2] = 0.0

    dsn = _norm_rows(ds_ref[0])
    dtn = _norm_rows(dt_ref[0])
    qrow = q_ref[0]
    trow = t_ref[0]
    qn = _norm_rows(qrow[:, :_D])
    tn = _norm_rows(trow[:, :_D])
    cs = jnp.sum(qrow[:, _D:], axis=-1)
    ct = jnp.sum(trow[:, _D:], axis=-1)
    rowsel = (lax.broadcasted_iota(jnp.int32, (_P, 1), 0) == p).astype(jnp.float32)
    vrow = jnp.sum(valid_ref[...] * rowsel, axis=0)
    vmask = (vrow > 0.5).astype(jnp.float32)

    inv_t = 1.0 / _TEMP
    dn = (((1,), (1,)), ((), ()))
    qb = (qn * inv_t).astype(jnp.bfloat16)
    tb = (tn * inv_t).astype(jnp.bfloat16)
    dsb = dsn.astype(jnp.bfloat16)
    dtb = dtn.astype(jnp.bfloat16)
    l_st = lax.dot_general(qb, dtb, dn,
                           preferred_element_type=jnp.float32)
    lse_st = jnp.log(jnp.sum(jnp.exp(l_st), axis=-1))
    l_ts = lax.dot_general(tb, dsb, dn,
                           preferred_element_type=jnp.float32)
    lse_ts = jnp.log(jnp.sum(jnp.exp(l_ts), axis=-1))
    gold = jnp.sum(qn * tn, axis=-1) * inv_t

    ce_pair = (lse_st + lse_ts - 2.0 * gold) * 0.5
    acc[0] += jnp.sum(ce_pair * vmask)
    acc[1] += jnp.sum(vmask)
    acc[2] += jnp.sum((_bce(cs, vmask) + _bce(ct, vmask)) * 0.5)

    @pl.when(p == _P - 1)
    def _():
        coarse = acc[0] / jnp.maximum(acc[1], 1.0)
        out_ref[0, 0] = _MATCH_W * coarse + _CONF_W * (acc[2] / float(_B))


_tc_loss = pl.pallas_call(
    _tc_body,
    grid=(_P,),
    in_specs=[
        pl.BlockSpec((1, _NC, _D), lambda p: (p, 0, 0)),
        pl.BlockSpec((1, _NC, _D), lambda p: (p, 0, 0)),
        pl.BlockSpec((1, _S, _W), lambda p: (p, 0, 0)),
        pl.BlockSpec((1, _S, _W), lambda p: (p, 0, 0)),
        pl.BlockSpec((_P, _S), lambda p: (0, 0)),
    ],
    out_specs=pl.BlockSpec((1, 1), lambda p: (0, 0), memory_space=pltpu.SMEM),
    out_shape=jax.ShapeDtypeStruct((1, 1), jnp.float32),
    scratch_shapes=[pltpu.SMEM((3,), jnp.float32)],
)


def kernel(desc_src, desc_tgt, conf_src, conf_tgt, src_idx, tgt_idx, valid):
    packed_s = (jnp.pad(desc_src, ((0, 0), (0, 0), (0, _W - _D)))
                + jnp.pad(conf_src[..., None], ((0, 0), (0, 0), (_D, _W - _D - 1))))
    packed_t = (jnp.pad(desc_tgt, ((0, 0), (0, 0), (0, _W - _D)))
                + jnp.pad(conf_tgt[..., None], ((0, 0), (0, 0), (_D, _W - _D - 1))))
    q_rows, t_rows = _make_sc_gather()(
        packed_s.reshape(_V, _W), packed_t.reshape(_V, _W),
        src_idx.astype(jnp.int32), tgt_idx.astype(jnp.int32))
    out = _tc_loss(desc_src, desc_tgt, q_rows, t_rows, valid)
    return out[0, 0]
